# Initial kernel scaffold; baseline (speedup 1.0000x reference)
import jax
import jax.numpy as jnp
from jax.experimental import pallas as pl


def kernel(gaussian_features, coordinates, pcd_coords):
    raise NotImplementedError("write your pallas kernel here")


# speedup vs baseline: 292.3287x; 292.3287x over previous
import functools

import jax
import jax.numpy as jnp
from jax import lax
from jax.experimental import pallas as pl
from jax.experimental.pallas import tpu as pltpu
from jax.experimental.pallas import tpu_sc as plsc

_C0 = 0.28209479177387814
_VOXEL = 64
_FAR = 100.0

_NS = 16
_NC = 2
_NW = _NC * _NS
_CHUNK = 128


def _sc_mask_kernel(n_pts, n_pcd):
    table_n = _VOXEL * _VOXEL * _VOXEL
    pts_per_tile = n_pts // _NW
    pcd_per_tile = n_pcd // _NS
    rows_pts = pts_per_tile // _CHUNK
    rows_pcd = pcd_per_tile // _CHUNK
    zslab = table_n // _NS

    mesh = plsc.VectorSubcoreMesh(core_axis_name="core", subcore_axis_name="subcore")

    @functools.partial(
        pl.kernel,
        mesh=mesh,
        out_type=jax.ShapeDtypeStruct((_NW, rows_pts, _CHUNK), jnp.float32),
        scratch_types=[
            pltpu.VMEM_SHARED((table_n,), jnp.float32),
            pltpu.VMEM((max(rows_pts, rows_pcd), _CHUNK), jnp.int32),
            pltpu.VMEM((rows_pts, _CHUNK), jnp.float32),
            pltpu.VMEM((pcd_per_tile,), jnp.int32),
            pltpu.VMEM((pcd_per_tile,), jnp.int32),
            pltpu.VMEM((pcd_per_tile,), jnp.int32),
            pltpu.VMEM((_CHUNK,), jnp.float32),
        ],
    )
    def mask_kernel(cx, cy, cz, px, py, pz, zeros_hbm, ones_hbm, mask_out,
                    table, keybuf, valbuf, xb, yb, zb, onesv):
        i32 = jnp.int32
        sid = lax.axis_index("subcore").astype(i32)
        wid = lax.axis_index("core").astype(i32) * i32(_NS) + sid

        pltpu.sync_copy(zeros_hbm, table.at[pl.ds(sid * i32(zslab), zslab)])
        pltpu.sync_copy(ones_hbm, onesv)

        pbase = sid * i32(pcd_per_tile)
        pltpu.sync_copy(px.at[pl.ds(pbase, pcd_per_tile)], xb)
        pltpu.sync_copy(py.at[pl.ds(pbase, pcd_per_tile)], yb)
        pltpu.sync_copy(pz.at[pl.ds(pbase, pcd_per_tile)], zb)
        plsc.subcore_barrier()

        @pl.loop(0, rows_pcd)
        def _scatter(r):
            for j in range(_CHUNK // _NS):
                off = r * i32(_CHUNK) + i32(j * _NS)
                k = (xb[pl.ds(off, _NS)] + yb[pl.ds(off, _NS)] * i32(_VOXEL)
                     + zb[pl.ds(off, _NS)] * i32(_VOXEL * _VOXEL))
                keybuf[r, pl.ds(j * _NS, _NS)] = k
            pltpu.sync_copy(onesv, table.at[keybuf.at[r]])

        plsc.subcore_barrier()

        cbase = wid * i32(pts_per_tile)
        pltpu.sync_copy(cx.at[pl.ds(cbase, pts_per_tile)], xb.at[pl.ds(0, pts_per_tile)])
        pltpu.sync_copy(cy.at[pl.ds(cbase, pts_per_tile)], yb.at[pl.ds(0, pts_per_tile)])
        pltpu.sync_copy(cz.at[pl.ds(cbase, pts_per_tile)], zb.at[pl.ds(0, pts_per_tile)])

        @pl.loop(0, rows_pts)
        def _gather(r):
            for j in range(_CHUNK // _NS):
                off = r * i32(_CHUNK) + i32(j * _NS)
                k = (xb[pl.ds(off, _NS)] + yb[pl.ds(off, _NS)] * i32(_VOXEL)
                     + zb[pl.ds(off, _NS)] * i32(_VOXEL * _VOXEL))
                keybuf[r, pl.ds(j * _NS, _NS)] = k
            pltpu.sync_copy(table.at[keybuf.at[r]], valbuf.at[r])

        pltpu.sync_copy(valbuf, mask_out.at[wid])

    return mask_kernel


def _stats_body(g_ref, mean_ref, scale_ref):
    x = g_ref[...]
    cnt = x.shape[0] * x.shape[1]
    s = jnp.sum(x)
    ss = jnp.sum(x * x)
    mean = s / cnt
    var = (ss - cnt * mean * mean) / (cnt - 1)
    mean_ref[0, 0] = mean
    scale_ref[0, 0] = (2.0 * _FAR / _VOXEL / 6.0) / jnp.sqrt(var)


def _map_body(mean_ref, scale_ref, g_ref, m_ref, o_ref):
    g = g_ref[...]
    mean = mean_ref[0, 0]
    dmscale = scale_ref[0, 0]

    dm = (g[0:3] - mean) * dmscale
    quat = g[3:7]
    sg = jax.nn.sigmoid(g[7:10])
    scale = sg * 2.0 * _FAR / _VOXEL
    opa = jax.nn.sigmoid(g[10:11] - 4.0)
    d1 = (jax.nn.sigmoid(g[11:14]) - 0.5) / _C0
    d2 = g[14:23] / 20.0
    d3 = g[23:38] / 40.0
    d4 = g[38:59] / 80.0

    qn = quat / jnp.sqrt(jnp.sum(quat * quat, axis=0, keepdims=True))
    r_, x_, y_, z_ = qn[0:1], qn[1:2], qn[2:3], qn[3:4]
    r00 = 1.0 - 2.0 * (y_ * y_ + z_ * z_)
    r01 = 2.0 * (x_ * y_ - r_ * z_)
    r02 = 2.0 * (x_ * z_ + r_ * y_)
    r10 = 2.0 * (x_ * y_ + r_ * z_)
    r11 = 1.0 - 2.0 * (x_ * x_ + z_ * z_)
    r12 = 2.0 * (y_ * z_ - r_ * x_)
    r20 = 2.0 * (x_ * z_ - r_ * y_)
    r21 = 2.0 * (y_ * z_ + r_ * x_)
    r22 = 1.0 - 2.0 * (x_ * x_ + y_ * y_)
    s0, s1, s2 = scale[0:1], scale[1:2], scale[2:3]
    l00, l01, l02 = r00 * s0, r01 * s1, r02 * s2
    l10, l11, l12 = r10 * s0, r11 * s1, r12 * s2
    l20, l21, l22 = r20 * s0, r21 * s1, r22 * s2
    c00 = l00 * l00 + l01 * l01 + l02 * l02
    c01 = l00 * l10 + l01 * l11 + l02 * l12
    c02 = l00 * l20 + l01 * l21 + l02 * l22
    c11 = l10 * l10 + l11 * l11 + l12 * l12
    c12 = l10 * l20 + l11 * l21 + l12 * l22
    c22 = l20 * l20 + l21 * l21 + l22 * l22

    maskrow = (m_ref[...] > 0.0).astype(jnp.float32)

    o_ref[...] = jnp.concatenate(
        [dm, quat, scale, opa, d1, d2, d3, d4,
         c00, c01, c02, c01, c11, c12, c02, c12, c22, maskrow], axis=0)


_MAP_BLK = 4096


def kernel(gaussian_features, coordinates, pcd_coords):
    ci = coordinates.astype(jnp.int32)
    pi = pcd_coords.astype(jnp.int32)
    with jax.enable_x64(False):
        return _kernel_x32(gaussian_features, ci, pi)


def _kernel_x32(gf, ci, pi):
    n = gf.shape[1]
    m = pi.shape[0]

    cx, cy, cz = ci[:, 0], ci[:, 1], ci[:, 2]
    px, py, pz = pi[:, 0], pi[:, 1], pi[:, 2]

    table_n = _VOXEL * _VOXEL * _VOXEL
    zeros_slab = jnp.zeros((table_n // _NS,), jnp.float32)
    ones_chunk = jnp.ones((_CHUNK,), jnp.float32)

    mask3d = _sc_mask_kernel(n, m)(cx, cy, cz, px, py, pz, zeros_slab,
                                   ones_chunk)
    mask = mask3d.reshape(1, n)

    gf3 = gf[0:3].reshape(n // 128 * 3, 128)
    mean, dmscale = pl.pallas_call(
        _stats_body,
        out_shape=[jax.ShapeDtypeStruct((1, 1), jnp.float32)] * 2,
        in_specs=[pl.BlockSpec(gf3.shape, lambda: (0, 0))],
        out_specs=[pl.BlockSpec(memory_space=pltpu.SMEM)] * 2,
    )(gf3)

    out = pl.pallas_call(
        _map_body,
        grid=(n // _MAP_BLK,),
        in_specs=[
            pl.BlockSpec((1, 1), lambda i: (0, 0), memory_space=pltpu.SMEM),
            pl.BlockSpec((1, 1), lambda i: (0, 0), memory_space=pltpu.SMEM),
            pl.BlockSpec((59, _MAP_BLK), lambda i: (0, i)),
            pl.BlockSpec((1, _MAP_BLK), lambda i: (0, i)),
        ],
        out_specs=pl.BlockSpec((69, _MAP_BLK), lambda i: (0, i)),
        out_shape=jax.ShapeDtypeStruct((69, n), jnp.float32),
    )(mean, dmscale, gf, mask)
    return out
